# baseline (speedup 1.0000x reference)
import functools

import jax
import jax.numpy as jnp
from jax import lax
from jax.experimental import pallas as pl
from jax.experimental.pallas import tpu as pltpu
from jax.experimental.pallas import tpu_sc as plsc

N = 4_000_000
NN = 2_000_000
NM = 1_800_000
NROWS = 2048
MAGIC = 12582912.0

NC, NS, L = 2, 16, 16
NW = NC * NS

UC = 14_336
UW = UC // 4
MOV_LO = NN // UC
MOV_HI = (NN + NM - 1) // UC
NMOV = MOV_HI - MOV_LO + 1
MASK_FRAME = NMOV * UC
MASK_LEFT_PAD = NN - MOV_LO * UC
NFULL = N // UC
TAIL = N - NFULL * UC
NCOPY = NFULL - NMOV


def _sc_quantize_body(pos_hbm, mwords_hbm, out_hbm, buf, mbuf):
    wid = lax.axis_index("s") * NC + lax.axis_index("c")
    iota4 = lax.iota(jnp.int32, L) * 4

    @pl.when(wid == 0)
    def _tail():
        pltpu.sync_copy(pos_hbm.at[pl.ds(NFULL * UC, TAIL)],
                        out_hbm.at[pl.ds(NFULL * UC, TAIL)])

    def copy_body(i, carry):
        j = wid + i * NW

        @pl.when(j < NCOPY)
        def _():
            base = jnp.where(j < MOV_LO, j * UC, (j + NMOV) * UC)
            pltpu.sync_copy(pos_hbm.at[pl.ds(base, UC)],
                            out_hbm.at[pl.ds(base, UC)])
        return carry

    lax.fori_loop(0, -(-NCOPY // NW), copy_body, 0)

    def mov_body(i, carry):
        k = wid + i * NW

        @pl.when(k < NMOV)
        def _():
            base = (MOV_LO + k) * UC
            pltpu.sync_copy(pos_hbm.at[pl.ds(base, UC)], buf)
            pltpu.sync_copy(mwords_hbm.at[pl.ds(k * UW, UW)], mbuf)

            def grp(g, c):
                w16 = mbuf[pl.ds(g * 16, 16)]
                xb = g * 64
                for b in range(4):
                    mb = (w16 >> (8 * b)) & 1 if b else w16 & 1
                    idx = xb + iota4 + b
                    x = plsc.load_gather(buf, [idx])
                    q = (x + MAGIC) - MAGIC
                    q = jnp.minimum(jnp.maximum(q, 0.0), float(NROWS - 1))
                    plsc.store_scatter(buf, [idx], q, mask=mb == 1)
                return c

            lax.fori_loop(0, UC // 64, grp, 0)
            pltpu.sync_copy(buf, out_hbm.at[pl.ds(base, UC)])
        return carry

    lax.fori_loop(0, -(-NMOV // NW), mov_body, 0)


@functools.lru_cache(maxsize=1)
def _build():
    mesh = plsc.VectorSubcoreMesh(core_axis_name="c", subcore_axis_name="s",
                                  num_cores=NC, num_subcores=NS)
    return pl.kernel(
        _sc_quantize_body,
        out_type=jax.ShapeDtypeStruct((N,), jnp.float32),
        mesh=mesh,
        scratch_types=[
            pltpu.VMEM((UC,), jnp.float32),
            pltpu.VMEM((UW,), jnp.int32),
        ],
        compiler_params=pltpu.CompilerParams(needs_layout_passes=False),
    )


def kernel(pos, mask):
    maskp = jnp.concatenate([
        jnp.zeros((MASK_LEFT_PAD,), jnp.uint8),
        mask.view(jnp.uint8),
        jnp.zeros((MASK_FRAME - MASK_LEFT_PAD - NM,), jnp.uint8),
    ])
    return _build()(pos, maskp.view(jnp.int32))


# speedup vs baseline: 2.3804x; 2.3804x over previous
import functools

import jax
import jax.numpy as jnp
from jax import lax
from jax.experimental import pallas as pl
from jax.experimental.pallas import tpu as pltpu
from jax.experimental.pallas import tpu_sc as plsc

N = 4_000_000
NN = 2_000_000
NM = 1_800_000
MAGIC = 12582912.0
QMAX = 2047.0

NC, NS, L = 2, 16, 16
NW = NC * NS

UC = 14_336
UW = UC // 4
NG = UC // 64
MOV_LO = NN // UC
MOV_HI = (NN + NM - 1) // UC
MASK_FRAME = (MOV_HI - MOV_LO + 1) * UC
MASK_LEFT_PAD = NN - MOV_LO * UC
NFULL = N // UC
TAIL = N - NFULL * UC
NV = -(-NFULL // NW)
NBUF = 3


def _sc_quantize_body(pos_hbm, mwords_hbm, out_hbm,
                      ibufs, obufs, mbufs, isems, osems):
    wid = lax.axis_index("s") * NC + lax.axis_index("c")

    def chunk_id(v):
        return wid + v * NW

    def is_mov(k):
        return jnp.logical_and(k >= MOV_LO, k <= MOV_HI)

    def in_descs(v):
        k = chunk_id(v)
        b = v % NBUF
        pos_d = pltpu.make_async_copy(
            pos_hbm.at[pl.ds(k * UC, UC)], ibufs[b], isems[b])
        m_d = pltpu.make_async_copy(
            mwords_hbm.at[pl.ds((k - MOV_LO) * UW, UW)], mbufs[b], isems[b])
        return k, pos_d, m_d

    def issue_in(v):
        k, pos_d, m_d = in_descs(v)

        @pl.when(k < NFULL)
        def _():
            pos_d.start()

            @pl.when(is_mov(k))
            def _():
                m_d.start()

    def wait_in(v):
        k, pos_d, m_d = in_descs(v)

        @pl.when(k < NFULL)
        def _():
            pos_d.wait()

            @pl.when(is_mov(k))
            def _():
                m_d.wait()

    def out_desc(v):
        k = chunk_id(v)
        b = v % NBUF
        return k, pltpu.make_async_copy(
            obufs[b], out_hbm.at[pl.ds(k * UC, UC)], osems[b])

    def compute(v):
        k = chunk_id(v)
        b = v % NBUF
        ibuf, obuf, mbuf = ibufs[b], obufs[b], mbufs[b]

        @pl.when(jnp.logical_and(k < NFULL, is_mov(k)))
        def _quant():
            def grp(g, c):
                w16 = mbuf[pl.ds(g * 16, 16)]
                for j in range(4):
                    sl = pl.ds(g * 64 + j * 16, 16)
                    x = ibuf[sl]
                    mb = (w16 << (31 - 8 * j)) < 0
                    q = jnp.minimum((x + MAGIC) - MAGIC, QMAX)
                    obuf[sl] = jnp.where(mb, q, x)
                return c

            lax.fori_loop(0, NG, grp, 0)

        @pl.when(jnp.logical_and(k < NFULL, jnp.logical_not(is_mov(k))))
        def _copy():
            def grp(g, c):
                for j in range(4):
                    sl = pl.ds(g * 64 + j * 16, 16)
                    obuf[sl] = ibuf[sl]
                return c

            lax.fori_loop(0, NG, grp, 0)

    def issue_out(v):
        k, d = out_desc(v)

        @pl.when(k < NFULL)
        def _():
            d.start()

    def wait_out(v):
        k, d = out_desc(v)

        @pl.when(k < NFULL)
        def _():
            d.wait()

    for v in range(NBUF):
        issue_in(v)
    for v in range(NV):
        if v >= NBUF:
            wait_out(v - NBUF)
        wait_in(v)
        compute(v)
        issue_out(v)
        if v + NBUF < NV:
            issue_in(v + NBUF)
    for v in range(NV - NBUF, NV):
        wait_out(v)

    @pl.when(wid == 0)
    def _tail():
        tb = ibufs[0].at[pl.ds(0, TAIL)]
        pltpu.sync_copy(pos_hbm.at[pl.ds(NFULL * UC, TAIL)], tb)
        pltpu.sync_copy(tb, out_hbm.at[pl.ds(NFULL * UC, TAIL)])


@functools.lru_cache(maxsize=1)
def _build():
    mesh = plsc.VectorSubcoreMesh(core_axis_name="c", subcore_axis_name="s",
                                  num_cores=NC, num_subcores=NS)
    return pl.kernel(
        _sc_quantize_body,
        out_type=jax.ShapeDtypeStruct((N,), jnp.float32),
        mesh=mesh,
        scratch_types=[
            [pltpu.VMEM((UC,), jnp.float32) for _ in range(NBUF)],
            [pltpu.VMEM((UC,), jnp.float32) for _ in range(NBUF)],
            [pltpu.VMEM((UW,), jnp.int32) for _ in range(NBUF)],
            [pltpu.SemaphoreType.DMA for _ in range(NBUF)],
            [pltpu.SemaphoreType.DMA for _ in range(NBUF)],
        ],
        compiler_params=pltpu.CompilerParams(needs_layout_passes=False),
    )


def kernel(pos, mask):
    maskp = jnp.concatenate([
        jnp.zeros((MASK_LEFT_PAD,), jnp.uint8),
        mask.view(jnp.uint8),
        jnp.zeros((MASK_FRAME - MASK_LEFT_PAD - NM,), jnp.uint8),
    ])
    mwords = (maskp.reshape(-1, 4, 16).transpose(0, 2, 1)
              .reshape(-1).view(jnp.int32))
    return _build()(pos, mwords)
